# Initial kernel scaffold; baseline (speedup 1.0000x reference)
import jax
import jax.numpy as jnp
from jax.experimental import pallas as pl


def kernel(x, l, params, pool_idx1, pool_idx2, pool_idx3):
    raise NotImplementedError("write your pallas kernel here")


# speedup vs baseline: 6.1446x; 6.1446x over previous
import jax
import jax.numpy as jnp
from jax.experimental import pallas as pl

_BIG = 2 ** 30


def _mm(a, b):
    return jax.lax.dot_general(a, b, (((1,), (0,)), ((), ())),
                               preferred_element_type=jnp.float32)


def _mmT(a, b):
    return jax.lax.dot_general(a, b, (((0,), (0,)), ((), ())),
                               preferred_element_type=jnp.float32)


def _mmh(a, b):
    return jax.lax.dot_general(a, b, (((1,), (0,)), ((), ())),
                               preferred_element_type=jnp.float32,
                               precision=jax.lax.Precision.HIGHEST)


def _lrelu(x):
    return jnp.where(x >= 0, x, 0.2 * x)


def _make_gather(nch, cw, c, tm):
    def gat(tblv, ii):
        iib = jnp.broadcast_to(ii, (c, tm))
        if nch == 1:
            loc = jnp.clip(iib, 0, cw - 1)
            return jnp.take_along_axis(tblv, loc, axis=1)
        g = jnp.zeros((c, tm), jnp.float32)
        for ch in range(nch):
            lo = ch * cw
            chunk = tblv[:, lo:lo + cw]
            loc = jnp.clip(iib - lo, 0, cw - 1)
            gc = jnp.take_along_axis(chunk, loc, axis=1)
            g = jnp.where((iib >= lo) & (iib < lo + cw), gc, g)
        return g
    return gat


def _bn_affine(s1, s2, gamma, beta, n):
    mu = jnp.sum(s1, axis=1, keepdims=True) / n
    var = jnp.sum(s2, axis=1, keepdims=True) / n - mu * mu
    sc = gamma[:, 0:1] / jnp.sqrt(var + 1e-5)
    return sc, beta[:, 0:1] - mu * sc


def _topk_mask(neg, kk, tm, N):
    lanes = jax.lax.broadcasted_iota(jnp.int32, (tm, N), 1)
    klanes = jax.lax.broadcasted_iota(jnp.int32, (tm, kk), 1)
    out = jnp.zeros((tm, kk), jnp.int32)
    for j in range(kk):
        m = jnp.max(neg, axis=1, keepdims=True)
        idx = jnp.min(jnp.where(neg == m, lanes, _BIG), axis=1)
        out = jnp.where(klanes == j, idx[:, None], out)
        neg = jnp.where(lanes == idx[:, None], -jnp.inf, neg)
    return out


def _knn_idx(q, p, kk, tm, nc=None):
    B, C, M = q.shape
    N = p.shape[2]

    def _norm(v):
        if nc is None:
            return jnp.sum(v * v, axis=0)
        s = v[0] * v[0]
        for c in range(1, nc):
            s = s + v[c] * v[c]
        return s

    def body(q_ref, p_ref, o_ref):
        qv = q_ref[0]
        pv = p_ref[0]
        qp = _mmT(qv, pv)
        qq = _norm(qv)[:, None]
        pn = _norm(pv)[None, :]
        neg = 2.0 * qp - qq - pn
        o_ref[0] = _topk_mask(neg, kk, tm, N)

    return pl.pallas_call(
        body,
        grid=(B, M // tm),
        in_specs=[
            pl.BlockSpec((1, C, tm), lambda b, t: (b, 0, t)),
            pl.BlockSpec((1, C, N), lambda b, t: (b, 0, 0)),
        ],
        out_specs=pl.BlockSpec((1, tm, kk), lambda b, t: (b, t, 0)),
        out_shape=jax.ShapeDtypeStruct((B, M, kk), jnp.int32),
    )(q, p)


def _pool_knn_idx(qT, p, kk, tm):
    B, M = qT.shape[:2]
    N = p.shape[2]

    def body(q_ref, p_ref, o_ref):
        qv = q_ref[0]
        pv = p_ref[0]
        d = jnp.zeros((tm, N), jnp.float32)
        for c in range(8):
            df = qv[:, c:c + 1] - pv[c:c + 1, :]
            d = d + df * df
        o_ref[0] = _topk_mask(-d, kk, tm, N)

    return pl.pallas_call(
        body,
        grid=(B, M // tm),
        in_specs=[
            pl.BlockSpec((1, tm, 8), lambda b, t: (b, t, 0)),
            pl.BlockSpec((1, 8, N), lambda b, t: (b, 0, 0)),
        ],
        out_specs=pl.BlockSpec((1, tm, kk), lambda b, t: (b, t, 0)),
        out_shape=jax.ShapeDtypeStruct((B, M, kk), jnp.int32),
    )(qT, p)


def _edge_conv(tbl, idxT, w1, g1, b1, w2, g2, b2, kk, tm, residual):
    B, C, N = tbl.shape
    C2a = w1.shape[0]
    C2b = w2.shape[0]
    tw = min(tm, 128)
    cw = min(N, 128)
    nch = (N + cw - 1) // cw
    n = float(B * N * kk)
    gat = _make_gather(nch, cw, C, tm)
    grid = (B, N // tm)

    tbl_full = pl.BlockSpec((1, C, N), lambda b, t: (b, 0, 0))
    tbl_tile = pl.BlockSpec((1, C, tm), lambda b, t: (b, 0, t))
    idx_spec = pl.BlockSpec((1, kk, tm), lambda b, t: (b, 0, t))

    def wsp(a):
        return pl.BlockSpec(a.shape, lambda b, t: (0,) * a.ndim)

    def stat_spec(c2):
        return pl.BlockSpec((c2, tw), lambda b, t: (0, 0))

    def first():
        return (pl.program_id(0) == 0) & (pl.program_id(1) == 0)

    def y1_stream(t_ref, c_ref, i_ref, w1_ref):
        tblv = t_ref[0]
        ctr = c_ref[0]
        idxv = i_ref[0]
        w1v = w1_ref[...]
        for k in range(kk):
            g = gat(tblv, idxv[k:k + 1, :])
            cat = jnp.concatenate([g - ctr, ctr], axis=0)
            yield _mm(w1v, cat), ctr

    def accum(acc1, acc2, y):
        ys = y * y
        for lg in range(tm // tw):
            sl = slice(lg * tw, (lg + 1) * tw)
            acc1 += y[:, sl]
            acc2 += ys[:, sl]
        return acc1, acc2

    def bodyA(t_ref, c_ref, i_ref, w1_ref, s1_ref, s2_ref):
        a1 = jnp.zeros((C2a, tw), jnp.float32)
        a2 = jnp.zeros((C2a, tw), jnp.float32)
        for y1, _ in y1_stream(t_ref, c_ref, i_ref, w1_ref):
            a1, a2 = accum(a1, a2, y1)

        @pl.when(first())
        def _():
            s1_ref[...] = jnp.zeros((C2a, tw), jnp.float32)
            s2_ref[...] = jnp.zeros((C2a, tw), jnp.float32)
        s1_ref[...] += a1
        s2_ref[...] += a2

    sA1, sA2 = pl.pallas_call(
        bodyA,
        grid=grid,
        in_specs=[tbl_full, tbl_tile, idx_spec, wsp(w1)],
        out_specs=[stat_spec(C2a), stat_spec(C2a)],
        out_shape=[jax.ShapeDtypeStruct((C2a, tw), jnp.float32)] * 2,
    )(tbl, tbl, idxT, w1)

    def bodyB(t_ref, c_ref, i_ref, w1_ref, w2_ref, g1_ref, b1_ref,
              sa1_ref, sa2_ref, s1_ref, s2_ref):
        sc1, sh1 = _bn_affine(sa1_ref[...], sa2_ref[...], g1_ref[...],
                              b1_ref[...], n)
        w2v = w2_ref[...]
        a1 = jnp.zeros((C2b, tw), jnp.float32)
        a2 = jnp.zeros((C2b, tw), jnp.float32)
        for y1, _ in y1_stream(t_ref, c_ref, i_ref, w1_ref):
            y2 = _mm(w2v, _lrelu(y1 * sc1 + sh1))
            a1, a2 = accum(a1, a2, y2)

        @pl.when(first())
        def _():
            s1_ref[...] = jnp.zeros((C2b, tw), jnp.float32)
            s2_ref[...] = jnp.zeros((C2b, tw), jnp.float32)
        s1_ref[...] += a1
        s2_ref[...] += a2

    sB1, sB2 = pl.pallas_call(
        bodyB,
        grid=grid,
        in_specs=[tbl_full, tbl_tile, idx_spec, wsp(w1), wsp(w2),
                  wsp(g1), wsp(b1), stat_spec(C2a), stat_spec(C2a)],
        out_specs=[stat_spec(C2b), stat_spec(C2b)],
        out_shape=[jax.ShapeDtypeStruct((C2b, tw), jnp.float32)] * 2,
    )(tbl, tbl, idxT, w1, w2, g1, b1, sA1, sA2)

    def bodyC(t_ref, c_ref, i_ref, w1_ref, w2_ref, g1_ref, b1_ref,
              g2_ref, b2_ref, sa1_ref, sa2_ref, sb1_ref, sb2_ref, *o_refs):
        sc1, sh1 = _bn_affine(sa1_ref[...], sa2_ref[...], g1_ref[...],
                              b1_ref[...], n)
        sc2, sh2 = _bn_affine(sb1_ref[...], sb2_ref[...], g2_ref[...],
                              b2_ref[...], n)
        w2v = w2_ref[...]
        acc = jnp.full((C2b, tm), -jnp.inf, jnp.float32)
        ctr_out = None
        for y1, ctr in y1_stream(t_ref, c_ref, i_ref, w1_ref):
            y2 = _mm(w2v, _lrelu(y1 * sc1 + sh1))
            acc = jnp.maximum(acc, _lrelu(y2 * sc2 + sh2))
            ctr_out = ctr
        o_refs[0][0] = acc
        if residual:
            o_refs[1][0] = jnp.maximum(acc + ctr_out, 0.0)

    out_specs = [pl.BlockSpec((1, C2b, tm), lambda b, t: (b, 0, t))]
    out_shape = [jax.ShapeDtypeStruct((B, C2b, N), jnp.float32)]
    if residual:
        out_specs = out_specs * 2
        out_shape = out_shape * 2

    outs = pl.pallas_call(
        bodyC,
        grid=grid,
        in_specs=[tbl_full, tbl_tile, idx_spec, wsp(w1), wsp(w2),
                  wsp(g1), wsp(b1), wsp(g2), wsp(b2),
                  stat_spec(C2a), stat_spec(C2a), stat_spec(C2b),
                  stat_spec(C2b)],
        out_specs=out_specs,
        out_shape=out_shape,
    )(tbl, tbl, idxT, w1, w2, g1, b1, g2, b2, sA1, sA2, sB1, sB2)
    if residual:
        return outs[0], outs[1]
    return outs[0]


def _conv_bn_act(x, w, gamma, beta, tsz, act=True, maxl=False, bias=None,
                 has_bn=True):
    B, Cin, S = x.shape
    C2 = w.shape[0]
    tw = min(tsz, 128)
    n = float(B * S)
    grid = (B, S // tsz)
    xspec = pl.BlockSpec((1, Cin, tsz), lambda b, t: (b, 0, t))

    def wsp(a):
        return pl.BlockSpec(a.shape, lambda b, t: (0,) * a.ndim)

    def stat_spec():
        return pl.BlockSpec((C2, tw), lambda b, t: (0, 0))

    stats = []
    if has_bn:
        def sbody(x_ref, w_ref, s1_ref, s2_ref):
            y = _mm(w_ref[...], x_ref[0])
            a1 = jnp.zeros((C2, tw), jnp.float32)
            a2 = jnp.zeros((C2, tw), jnp.float32)
            ys = y * y
            for lg in range(tsz // tw):
                sl = slice(lg * tw, (lg + 1) * tw)
                a1 += y[:, sl]
                a2 += ys[:, sl]

            @pl.when((pl.program_id(0) == 0) & (pl.program_id(1) == 0))
            def _():
                s1_ref[...] = jnp.zeros((C2, tw), jnp.float32)
                s2_ref[...] = jnp.zeros((C2, tw), jnp.float32)
            s1_ref[...] += a1
            s2_ref[...] += a2

        stats = list(pl.pallas_call(
            sbody,
            grid=grid,
            in_specs=[xspec, wsp(w)],
            out_specs=[stat_spec(), stat_spec()],
            out_shape=[jax.ShapeDtypeStruct((C2, tw), jnp.float32)] * 2,
        )(x, w))

    extra = []
    extra_specs = []
    if has_bn:
        extra = [stats[0], stats[1], gamma, beta]
        extra_specs = [stat_spec(), stat_spec(), wsp(gamma), wsp(beta)]
    if bias is not None:
        extra.append(bias)
        extra_specs.append(wsp(bias))

    def abody(x_ref, w_ref, *rest):
        o_ref = rest[-1]
        y = _mm(w_ref[...], x_ref[0])
        i = 0
        if has_bn:
            sc, sh = _bn_affine(rest[0][...], rest[1][...], rest[2][...],
                                rest[3][...], n)
            y = y * sc + sh
            i = 4
        if bias is not None:
            y = y + rest[i][...][:, 0:1]
        if act:
            y = _lrelu(y)
        if maxl:
            cur = jnp.broadcast_to(jnp.max(y, axis=1, keepdims=True),
                                   (C2, tw))
            prev = o_ref[0]
            o_ref[0] = jnp.where(pl.program_id(1) == 0, cur,
                                 jnp.maximum(prev, cur))
        else:
            o_ref[0] = y

    if maxl:
        ospec = pl.BlockSpec((1, C2, tw), lambda b, t: (b, 0, 0))
        oshape = jax.ShapeDtypeStruct((B, C2, tw), jnp.float32)
    else:
        ospec = pl.BlockSpec((1, C2, tsz), lambda b, t: (b, 0, t))
        oshape = jax.ShapeDtypeStruct((B, C2, S), jnp.float32)

    return pl.pallas_call(
        abody,
        grid=grid,
        in_specs=[xspec, wsp(w)] + extra_specs,
        out_specs=ospec,
        out_shape=oshape,
    )(x, w, *extra)


def _gather_max(tbl, idxT, kk, tm):
    B, C, N = tbl.shape
    M = idxT.shape[2]
    cw = min(N, 128)
    nch = (N + cw - 1) // cw
    gat = _make_gather(nch, cw, C, tm)

    def body(t_ref, i_ref, o_ref):
        tblv = t_ref[0]
        idxv = i_ref[0]
        acc = jnp.full((C, tm), -jnp.inf, jnp.float32)
        for k in range(kk):
            acc = jnp.maximum(acc, gat(tblv, idxv[k:k + 1, :]))
        o_ref[0] = acc

    return pl.pallas_call(
        body,
        grid=(B, M // tm),
        in_specs=[
            pl.BlockSpec((1, C, N), lambda b, t: (b, 0, 0)),
            pl.BlockSpec((1, kk, tm), lambda b, t: (b, 0, t)),
        ],
        out_specs=pl.BlockSpec((1, C, tm), lambda b, t: (b, 0, t)),
        out_shape=jax.ShapeDtypeStruct((B, C, M), jnp.float32),
    )(tbl, idxT)


def _gather_cols(tbl, sel8, tm):
    B, C, N = tbl.shape
    M = sel8.shape[1]
    cw = min(N, 128)
    nch = (N + cw - 1) // cw
    gat = _make_gather(nch, cw, C, tm)

    def body(t_ref, s_ref, o_ref):
        o_ref[0] = gat(t_ref[0], s_ref[0:1, :])

    return pl.pallas_call(
        body,
        grid=(B, M // tm),
        in_specs=[
            pl.BlockSpec((1, C, N), lambda b, t: (b, 0, 0)),
            pl.BlockSpec((8, tm), lambda b, t: (0, t)),
        ],
        out_specs=pl.BlockSpec((1, C, tm), lambda b, t: (b, 0, t)),
        out_shape=jax.ShapeDtypeStruct((B, C, M), jnp.float32),
    )(tbl, sel8)


def _unpool(nd, nuT, ftT, tm):
    B, _, Md = nd.shape
    Mu = nuT.shape[1]
    C = ftT.shape[2]

    def body(u_ref, d_ref, f_ref, o_ref):
        qv = u_ref[0]
        pv = d_ref[0]
        d = jnp.zeros((tm, Md), jnp.float32)
        for c in range(8):
            df = qv[:, c:c + 1] - pv[c:c + 1, :]
            d = d + df * df
        lanes = jax.lax.broadcasted_iota(jnp.int32, (tm, Md), 1)
        m = jnp.min(d, axis=1, keepdims=True)
        idx = jnp.min(jnp.where(d == m, lanes, _BIG), axis=1)
        oh = (lanes == idx[:, None]).astype(jnp.float32)
        o_ref[0] = _mmh(oh, f_ref[0])

    return pl.pallas_call(
        body,
        grid=(B, Mu // tm),
        in_specs=[
            pl.BlockSpec((1, tm, 8), lambda b, t: (b, t, 0)),
            pl.BlockSpec((1, 8, Md), lambda b, t: (b, 0, 0)),
            pl.BlockSpec((1, Md, C), lambda b, t: (b, 0, 0)),
        ],
        out_specs=pl.BlockSpec((1, tm, C), lambda b, t: (b, t, 0)),
        out_shape=jax.ShapeDtypeStruct((B, Mu, C), jnp.float32),
    )(nuT, nd, ftT)


def _transform(xp, tpad):
    B, _, N = xp.shape

    def body(t_ref, x_ref, o_ref):
        o_ref[0] = _mmT(t_ref[0], x_ref[0])

    return pl.pallas_call(
        body,
        grid=(B,),
        in_specs=[
            pl.BlockSpec((1, 8, 8), lambda b: (b, 0, 0)),
            pl.BlockSpec((1, 8, N), lambda b: (b, 0, 0)),
        ],
        out_specs=pl.BlockSpec((1, 8, N), lambda b: (b, 0, 0)),
        out_shape=jax.ShapeDtypeStruct((B, 8, N), jnp.float32),
    )(tpad, xp)


def _bnv(v):
    return jnp.broadcast_to(v[:, None], (v.shape[0], 8))


def _cat_w1(w, cin):
    wa = w[:, :cin]
    wb = w[:, cin:]
    if cin < 8:
        pad = ((0, 0), (0, 8 - cin))
        wa = jnp.pad(wa, pad)
        wb = jnp.pad(wb, pad)
    return jnp.concatenate([wa, wb], axis=1)


def _T(idx):
    return jnp.swapaxes(idx, 1, 2)


def kernel(x, l, params, pool_idx1, pool_idx2, pool_idx3):
    p = params
    B, _, N = x.shape
    K = 40

    def ec_params(name):
        return (_bnv(p[name + '_1_g']), _bnv(p[name + '_1_b']),
                p[name + '_2_w'], _bnv(p[name + '_2_g']),
                _bnv(p[name + '_2_b']))

    xp = jnp.pad(x, ((0, 0), (0, 5), (0, 0)))

    idx_t = _knn_idx(xp, xp, K, 256, nc=3)
    tf = _edge_conv(xp, _T(idx_t), _cat_w1(p['t1_w'], 3),
                    _bnv(p['t1_g']), _bnv(p['t1_b']),
                    p['t2_w'], _bnv(p['t2_g']), _bnv(p['t2_b']),
                    K, 256, False)
    t3m = _conv_bn_act(tf, p['t3_w'], _bnv(p['t3_g']), _bnv(p['t3_b']),
                       256, maxl=True)
    xt = jnp.transpose(t3m[:, :, 0]).reshape(1, 1024, B)
    h = _conv_bn_act(xt, p['tl1_w'], _bnv(p['tl1_g']), _bnv(p['tl1_b']), 8)
    h = _conv_bn_act(h, p['tl2_w'], _bnv(p['tl2_g']), _bnv(p['tl2_b']), 8)
    ttw = jnp.pad(p['tt_w'], ((0, 7), (0, 0)))
    ttb = jnp.pad(p['tt_b'], (0, 7))
    tt = _conv_bn_act(h, ttw, None, None, 8, act=False, has_bn=False,
                      bias=_bnv(ttb))
    t = jnp.transpose(tt[0, :9, :]).reshape(B, 3, 3)
    tpad = jnp.pad(t, ((0, 0), (0, 5), (0, 5)))
    node0 = _transform(xp, tpad)

    idx0 = _knn_idx(node0, node0, K, 256, nc=3)
    x0 = _edge_conv(node0, _T(idx0), _cat_w1(p['ec0_1_w'], 3),
                    *ec_params('ec0'),
                    kk=K, tm=256, residual=False)
    xt0 = _conv_bn_act(x0, p['pn0_w'], _bnv(p['pn0_g']), _bnv(p['pn0_b']),
                       256, maxl=True)[:, :, 0]

    sel1 = jnp.broadcast_to(pool_idx1.astype(jnp.int32)[None, :], (8, 512))
    node1 = _gather_cols(node0, sel1, 256)
    nidx1 = _pool_knn_idx(jnp.swapaxes(node1, 1, 2), node0, K, 256)
    n1f = _gather_max(x0, _T(nidx1), K, 256)
    idx1 = _knn_idx(n1f, n1f, K, 256)
    x1, x1r = _edge_conv(n1f, _T(idx1), _cat_w1(p['ec1_1_w'], 64),
                         *ec_params('ec1'),
                         kk=K, tm=256, residual=True)
    xt1 = _conv_bn_act(x1, p['pn1_w'], _bnv(p['pn1_g']), _bnv(p['pn1_b']),
                       256, maxl=True)[:, :, 0]

    sel2 = jnp.broadcast_to(pool_idx2.astype(jnp.int32)[None, :], (8, 128))
    node2 = _gather_cols(node1, sel2, 128)
    nidx2 = _pool_knn_idx(jnp.swapaxes(node2, 1, 2), node1, K, 128)
    n2f = _gather_max(x1r, _T(nidx2), K, 128)
    idx2 = _knn_idx(n2f, n2f, K, 128)
    x2, x2r = _edge_conv(n2f, _T(idx2), _cat_w1(p['ec2_1_w'], 64),
                         *ec_params('ec2'),
                         kk=K, tm=128, residual=True)
    xt2 = _conv_bn_act(x2, p['pn2_w'], _bnv(p['pn2_g']), _bnv(p['pn2_b']),
                       128, maxl=True)[:, :, 0]

    sel3 = jnp.broadcast_to(pool_idx3.astype(jnp.int32)[None, :], (8, 32))
    node3 = _gather_cols(node2, sel3, 32)
    nidx3 = _pool_knn_idx(jnp.swapaxes(node3, 1, 2), node2, K, 32)
    n3f = _gather_max(x2r, _T(nidx3), K, 32)
    idx3 = _knn_idx(n3f, n3f, K // 2, 32)
    x3, x3r = _edge_conv(n3f, _T(idx3), _cat_w1(p['ec3_1_w'], 64),
                         *ec_params('ec3'),
                         kk=K // 2, tm=32, residual=True)
    xt3 = _conv_bn_act(x3, p['pn3_w'], _bnv(p['pn3_g']), _bnv(p['pn3_b']),
                       32, maxl=True)[:, :, 0]

    g = jnp.maximum(jnp.maximum(xt0, xt1), jnp.maximum(xt2, xt3))
    lT = jnp.transpose(l).reshape(1, 16, B)
    lv = _conv_bn_act(lT, p['label_w'], _bnv(p['label_g']),
                      _bnv(p['label_b']), 8)
    gl = jnp.concatenate([g, jnp.transpose(lv[0])], axis=1)
    h = jnp.concatenate(
        [jnp.broadcast_to(gl[:, :, None], (B, 1088, 32)), x3r], axis=1)
    h = _conv_bn_act(h, p['pn4_w'], _bnv(p['pn4_g']), _bnv(p['pn4_b']), 32)

    u = _unpool(node3, jnp.swapaxes(node2, 1, 2),
                jnp.swapaxes(h, 1, 2), 128)
    h = jnp.concatenate([jnp.swapaxes(u, 1, 2), x2r], axis=1)
    h = _conv_bn_act(h, p['pn5_w'], _bnv(p['pn5_g']), _bnv(p['pn5_b']), 128)
    u = _unpool(node2, jnp.swapaxes(node1, 1, 2),
                jnp.swapaxes(h, 1, 2), 256)
    h = jnp.concatenate([jnp.swapaxes(u, 1, 2), x1r], axis=1)
    h = _conv_bn_act(h, p['pn6_w'], _bnv(p['pn6_g']), _bnv(p['pn6_b']), 256)
    u = _unpool(node1, jnp.swapaxes(node0, 1, 2),
                jnp.swapaxes(h, 1, 2), 256)
    h = jnp.concatenate([jnp.swapaxes(u, 1, 2), x0], axis=1)
    h = _conv_bn_act(h, p['pn7_w'], _bnv(p['pn7_g']), _bnv(p['pn7_b']), 256)
    c8 = jnp.pad(p['c8_w'], ((0, 14), (0, 0)))
    out = _conv_bn_act(h, c8, None, None, 256, act=False, has_bn=False)
    return out[:, :50, :]
